# Initial kernel scaffold; baseline (speedup 1.0000x reference)
import jax
import jax.numpy as jnp
from jax.experimental import pallas as pl


def kernel(x_paper, x_author, edge_index_writes, edge_index_cites, edge_index_rev_writes, Wl_writes_0, bl_writes_0, Wr_writes_0, Wl_cites_0, bl_cites_0, Wr_cites_0, Wl_rev_writes_0, bl_rev_writes_0, Wr_rev_writes_0, Wl_writes_1, bl_writes_1, Wr_writes_1, Wl_cites_1, bl_cites_1, Wr_cites_1, Wl_rev_writes_1, bl_rev_writes_1, Wr_rev_writes_1):
    raise NotImplementedError("write your pallas kernel here")


# speedup vs baseline: 1.8414x; 1.8414x over previous
import functools

import jax
import jax.numpy as jnp
from jax import lax
from jax.experimental import pallas as pl
from jax.experimental.pallas import tpu as pltpu
from jax.experimental.pallas import tpu_sc as plsc

NN = 10000
DD = 128
EE = 160000
NC = 2
NS = 16
NWORK = NC * NS
SUB = 128
EW = 5120
E_PAD = EW * NWORK
CHUNK_SUBS = 8
N_CHUNK = EW // (SUB * CHUNK_SUBS)
N_PAD = 10112
ZROWS = N_PAD // NS
FROWS = 632
FLAST = NN - (NS - 1) * FROWS
CW = 16
RT = 1000


def _seg_mesh():
    return plsc.VectorSubcoreMesh(core_axis_name="c", subcore_axis_name="s")


def _hbm_to_spmem(hbm, spm, base, total, stage):
    off = 0
    while off < total:
        L = min(SUB, total - off)
        pltpu.sync_copy(hbm.at[pl.ds(base + off, L)], stage.at[pl.ds(0, L)])
        pltpu.sync_copy(stage.at[pl.ds(0, L)], spm.at[pl.ds(base + off, L)])
        off += L


def _spmem_to_hbm(spm, hbm, c, base, total, stage):
    off = 0
    while off < total:
        L = min(SUB, total - off)
        pltpu.sync_copy(spm.at[pl.ds(base + off, L)], stage.at[pl.ds(0, L)])
        pltpu.sync_copy(stage.at[pl.ds(0, L)], hbm.at[c, pl.ds(base + off, L)])
        off += L


def _seg_nocount_body(x_hbm, src_hbm, dst_hbm, zf_hbm,
                      s_out, idx_s, idx_d, rows, acc, sem):
    c = lax.axis_index("c")
    s = lax.axis_index("s")
    wid = c * NS + s

    z0 = s * ZROWS
    _hbm_to_spmem(zf_hbm, acc, z0, ZROWS, rows)
    plsc.subcore_barrier()

    row_base = wid * (EW // SUB)

    def chunk(j, carry):
        r0 = row_base + j * CHUNK_SUBS
        pltpu.sync_copy(src_hbm.at[pl.ds(r0, CHUNK_SUBS)], idx_s)
        pltpu.sync_copy(dst_hbm.at[pl.ds(r0, CHUNK_SUBS)], idx_d)
        for i in range(CHUNK_SUBS):
            pltpu.async_copy(x_hbm.at[idx_s.at[i]], rows, sem).wait()
            pltpu.sync_copy(rows, acc.at[idx_d.at[i]], add=True)
        return carry

    lax.fori_loop(0, N_CHUNK, chunk, 0)

    plsc.subcore_barrier()
    f0 = s * FROWS

    @pl.when(s < NS - 1)
    def _flush_main():
        _spmem_to_hbm(acc, s_out, c, f0, FROWS, rows)

    @pl.when(s == NS - 1)
    def _flush_last():
        _spmem_to_hbm(acc, s_out, c, f0, FLAST, rows)


_seg_nocount = functools.partial(
    pl.kernel,
    mesh=_seg_mesh(),
    out_type=jax.ShapeDtypeStruct((NC, NN, DD), jnp.float32),
    scratch_types=[
        pltpu.VMEM((CHUNK_SUBS, SUB), jnp.int32),
        pltpu.VMEM((CHUNK_SUBS, SUB), jnp.int32),
        pltpu.VMEM((SUB, DD), jnp.float32),
        pltpu.VMEM_SHARED((N_PAD, DD), jnp.float32),
        pltpu.SemaphoreType.DMA,
    ],
)(_seg_nocount_body)


def _inv_deg(c_ref):
    c = c_ref[0, :, 0:1] + c_ref[1, :, 0:1]
    return 1.0 / jnp.maximum(c, 1.0)


def _l0_paper_body(sw_ref, sc_ref, cw_ref, cc_ref, xp_ref,
                   wlw_ref, wlc_ref, wrw_ref, wrc_ref, b_ref, o_ref):
    aggw = (sw_ref[0] + sw_ref[1]) * _inv_deg(cw_ref)
    aggc = (sc_ref[0] + sc_ref[1]) * _inv_deg(cc_ref)
    h = jnp.dot(aggw, wlw_ref[...], preferred_element_type=jnp.float32)
    h = h + jnp.dot(aggc, wlc_ref[...], preferred_element_type=jnp.float32)
    h = h + jnp.dot(xp_ref[...], wrw_ref[...] + wrc_ref[...],
                    preferred_element_type=jnp.float32)
    o_ref[...] = jnp.maximum(h + b_ref[...], 0.0)


def _l0_author_body(sr_ref, cr_ref, xa_ref, wlr_ref, wrr_ref, b_ref, o_ref):
    aggr = (sr_ref[0] + sr_ref[1]) * _inv_deg(cr_ref)
    h = jnp.dot(aggr, wlr_ref[...], preferred_element_type=jnp.float32)
    h = h + jnp.dot(xa_ref[...], wrr_ref[...], preferred_element_type=jnp.float32)
    o_ref[...] = jnp.maximum(h + b_ref[...], 0.0)


def _mm_body(x_ref, w_ref, o_ref):
    o_ref[...] = jnp.dot(x_ref[...], w_ref[...], preferred_element_type=jnp.float32)


def _l1_paper_body(sw_ref, sc_ref, cw_ref, cc_ref, hp_ref,
                   wrw_ref, wrc_ref, b_ref, o_ref):
    aggw = (sw_ref[0] + sw_ref[1]) * _inv_deg(cw_ref)
    aggc = (sc_ref[0] + sc_ref[1]) * _inv_deg(cc_ref)
    h = aggw + aggc + jnp.dot(hp_ref[...], wrw_ref[...] + wrc_ref[...],
                              preferred_element_type=jnp.float32)
    o_ref[...] = h + b_ref[...]


def _part_spec(d):
    return pl.BlockSpec((NC, RT, d), lambda i: (0, i, 0))


def _row_spec(d):
    return pl.BlockSpec((RT, d), lambda i: (i, 0))


def _full_spec(r, d):
    return pl.BlockSpec((r, d), lambda i: (0, 0))


_GRID = (NN // RT,)

_l0_paper = pl.pallas_call(
    _l0_paper_body,
    grid=_GRID,
    in_specs=[_part_spec(DD), _part_spec(DD), _part_spec(DD), _part_spec(DD),
              _row_spec(DD), _full_spec(DD, 256), _full_spec(DD, 256),
              _full_spec(DD, 256), _full_spec(DD, 256), _full_spec(1, 256)],
    out_specs=_row_spec(256),
    out_shape=jax.ShapeDtypeStruct((NN, 256), jnp.float32),
)

_l0_author = pl.pallas_call(
    _l0_author_body,
    grid=_GRID,
    in_specs=[_part_spec(DD), _part_spec(DD), _row_spec(DD),
              _full_spec(DD, 256), _full_spec(DD, 256), _full_spec(1, 256)],
    out_specs=_row_spec(256),
    out_shape=jax.ShapeDtypeStruct((NN, 256), jnp.float32),
)

_mm_256_128 = pl.pallas_call(
    _mm_body,
    grid=_GRID,
    in_specs=[_row_spec(256), _full_spec(256, DD)],
    out_specs=_row_spec(DD),
    out_shape=jax.ShapeDtypeStruct((NN, DD), jnp.float32),
)

_l1_paper = pl.pallas_call(
    _l1_paper_body,
    grid=_GRID,
    in_specs=[_part_spec(DD), _part_spec(DD), _part_spec(DD), _part_spec(DD),
              _row_spec(256), _full_spec(256, DD), _full_spec(256, DD),
              _full_spec(1, DD)],
    out_specs=_row_spec(DD),
    out_shape=jax.ShapeDtypeStruct((NN, DD), jnp.float32),
)


def _prep_edges(ei):
    pad = E_PAD - EE
    src = jnp.concatenate([ei[0], jnp.zeros((pad,), jnp.int32)])
    dst = jnp.concatenate([ei[1], jnp.full((pad,), NN, jnp.int32)])
    return src.reshape(E_PAD // SUB, SUB), dst.reshape(E_PAD // SUB, SUB)


def kernel(x_paper, x_author, edge_index_writes, edge_index_cites,
           edge_index_rev_writes,
           Wl_writes_0, bl_writes_0, Wr_writes_0,
           Wl_cites_0, bl_cites_0, Wr_cites_0,
           Wl_rev_writes_0, bl_rev_writes_0, Wr_rev_writes_0,
           Wl_writes_1, bl_writes_1, Wr_writes_1,
           Wl_cites_1, bl_cites_1, Wr_cites_1,
           Wl_rev_writes_1, bl_rev_writes_1, Wr_rev_writes_1):
    zf = jnp.zeros((N_PAD, DD), jnp.float32)
    ones_n = jnp.ones((NN, DD), jnp.float32)

    src_w, dst_w = _prep_edges(edge_index_writes)
    src_c, dst_c = _prep_edges(edge_index_cites)
    src_r, dst_r = _prep_edges(edge_index_rev_writes)

    s_w0 = _seg_nocount(x_author, src_w, dst_w, zf)
    s_c0 = _seg_nocount(x_paper, src_c, dst_c, zf)
    s_r0 = _seg_nocount(x_paper, src_r, dst_r, zf)
    cnt_w = _seg_nocount(ones_n, src_w, dst_w, zf)
    cnt_c = _seg_nocount(ones_n, src_c, dst_c, zf)
    cnt_r = _seg_nocount(ones_n, src_r, dst_r, zf)

    b_p0 = (bl_writes_0 + bl_cites_0).reshape(1, 256)
    b_a0 = bl_rev_writes_0.reshape(1, 256)
    h_p = _l0_paper(s_w0, s_c0, cnt_w, cnt_c, x_paper,
                    Wl_writes_0, Wl_cites_0, Wr_writes_0, Wr_cites_0, b_p0)
    h_a = _l0_author(s_r0, cnt_r, x_author, Wl_rev_writes_0, Wr_rev_writes_0,
                     b_a0)

    a1 = _mm_256_128(h_a, Wl_writes_1)
    pc = _mm_256_128(h_p, Wl_cites_1)
    s_w1 = _seg_nocount(a1, src_w, dst_w, zf)
    s_c1 = _seg_nocount(pc, src_c, dst_c, zf)

    b_p1 = (bl_writes_1 + bl_cites_1).reshape(1, DD)
    out = _l1_paper(s_w1, s_c1, cnt_w, cnt_c, h_p,
                    Wr_writes_1, Wr_cites_1, b_p1)
    return out


# speedup vs baseline: 2.8844x; 1.5665x over previous
import functools

import jax
import jax.numpy as jnp
from jax import lax
from jax.experimental import pallas as pl
from jax.experimental.pallas import tpu as pltpu
from jax.experimental.pallas import tpu_sc as plsc

NN = 10000
DD = 128
EE = 160000
NC = 2
NS = 16
NWORK = NC * NS
SUB = 128
EW = 5120
E_PAD = EW * NWORK
CHUNK_SUBS = 8
N_CHUNK = EW // (SUB * CHUNK_SUBS)
N_PAD = 10112
ZROWS = N_PAD // NS
FROWS = 632
FLAST = NN - (NS - 1) * FROWS
CW = 16
RT = 1000


def _seg_mesh():
    return plsc.VectorSubcoreMesh(core_axis_name="c", subcore_axis_name="s")


def _hbm_to_spmem(hbm, spm, base, total, stage):
    off = 0
    while off < total:
        L = min(SUB, total - off)
        pltpu.sync_copy(hbm.at[pl.ds(base + off, L)], stage.at[pl.ds(0, L)])
        pltpu.sync_copy(stage.at[pl.ds(0, L)], spm.at[pl.ds(base + off, L)])
        off += L


def _spmem_to_hbm(spm, hbm, c, base, total, stage):
    off = 0
    while off < total:
        L = min(SUB, total - off)
        pltpu.sync_copy(spm.at[pl.ds(base + off, L)], stage.at[pl.ds(0, L)])
        pltpu.sync_copy(stage.at[pl.ds(0, L)], hbm.at[c, pl.ds(base + off, L)])
        off += L


def _seg_nocount_body(x_hbm, src_hbm, dst_hbm, zf_hbm,
                      s_out, idx_s, idx_d, rows0, rows1, acc,
                      sg0, sg1, ss0, ss1):
    c = lax.axis_index("c")
    s = lax.axis_index("s")
    wid = c * NS + s

    z0 = s * ZROWS
    _hbm_to_spmem(zf_hbm, acc, z0, ZROWS, rows0)
    plsc.subcore_barrier()

    row_base = wid * (EW // SUB)
    rows = (rows0, rows1)
    sg = (sg0, sg1)
    ss = (ss0, ss1)

    def chunk(j, carry):
        r0 = row_base + j * CHUNK_SUBS
        pltpu.sync_copy(src_hbm.at[pl.ds(r0, CHUNK_SUBS)], idx_s)
        pltpu.sync_copy(dst_hbm.at[pl.ds(r0, CHUNK_SUBS)], idx_d)
        gd = [None, None]
        sd = [None, None]
        gd[0] = pltpu.async_copy(x_hbm.at[idx_s.at[0]], rows[0], sg[0])
        for i in range(CHUNK_SUBS):
            cb = i % 2
            gd[cb].wait()
            if i < CHUNK_SUBS - 1:
                if i >= 1:
                    sd[1 - cb].wait()
                gd[1 - cb] = pltpu.async_copy(
                    x_hbm.at[idx_s.at[i + 1]], rows[1 - cb], sg[1 - cb])
            sd[cb] = pltpu.async_copy(
                rows[cb], acc.at[idx_d.at[i]], ss[cb], add=True)
        sd[0].wait()
        sd[1].wait()
        return carry

    lax.fori_loop(0, N_CHUNK, chunk, 0)

    plsc.subcore_barrier()
    f0 = s * FROWS

    @pl.when(s < NS - 1)
    def _flush_main():
        _spmem_to_hbm(acc, s_out, c, f0, FROWS, rows0)

    @pl.when(s == NS - 1)
    def _flush_last():
        _spmem_to_hbm(acc, s_out, c, f0, FLAST, rows0)


def _deg_body(ones_hbm, dstw_hbm, dstc_hbm, dstr_hbm, zf_hbm,
              w_out, c_out, r_out, idx_d, ones_v, acc, sem):
    c = lax.axis_index("c")
    s = lax.axis_index("s")
    wid = c * NS + s
    z0 = s * ZROWS
    f0 = s * FROWS
    row_base = wid * (EW // SUB)

    pltpu.sync_copy(ones_hbm.at[pl.ds(0, SUB)], ones_v)

    for dst_hbm, out_hbm in ((dstw_hbm, w_out), (dstc_hbm, c_out),
                             (dstr_hbm, r_out)):
        _hbm_to_spmem(zf_hbm, acc, z0, ZROWS, ones_v)
        pltpu.sync_copy(ones_hbm.at[pl.ds(0, SUB)], ones_v)
        plsc.subcore_barrier()

        def chunk(j, carry, dst_hbm=dst_hbm):
            r0 = row_base + j * CHUNK_SUBS
            pltpu.sync_copy(dst_hbm.at[pl.ds(r0, CHUNK_SUBS)], idx_d)
            ds = [pltpu.async_copy(ones_v, acc.at[idx_d.at[i]], sem, add=True)
                  for i in range(CHUNK_SUBS)]
            for d in ds:
                d.wait()
            return carry

        lax.fori_loop(0, N_CHUNK, chunk, 0)
        plsc.subcore_barrier()

        @pl.when(s < NS - 1)
        def _flush_main(out_hbm=out_hbm):
            _spmem_to_hbm(acc, out_hbm, c, f0, FROWS, ones_v)

        @pl.when(s == NS - 1)
        def _flush_last(out_hbm=out_hbm):
            _spmem_to_hbm(acc, out_hbm, c, f0, FLAST, ones_v)
        plsc.subcore_barrier()


_seg_nocount = functools.partial(
    pl.kernel,
    mesh=_seg_mesh(),
    out_type=jax.ShapeDtypeStruct((NC, NN, DD), jnp.float32),
    scratch_types=[
        pltpu.VMEM((CHUNK_SUBS, SUB), jnp.int32),
        pltpu.VMEM((CHUNK_SUBS, SUB), jnp.int32),
        pltpu.VMEM((SUB, DD), jnp.float32),
        pltpu.VMEM((SUB, DD), jnp.float32),
        pltpu.VMEM_SHARED((N_PAD, DD), jnp.float32),
        pltpu.SemaphoreType.DMA,
        pltpu.SemaphoreType.DMA,
        pltpu.SemaphoreType.DMA,
        pltpu.SemaphoreType.DMA,
    ],
)(_seg_nocount_body)

_deg_counts = functools.partial(
    pl.kernel,
    mesh=_seg_mesh(),
    out_type=[
        jax.ShapeDtypeStruct((NC, NN, DD), jnp.float32),
        jax.ShapeDtypeStruct((NC, NN, DD), jnp.float32),
        jax.ShapeDtypeStruct((NC, NN, DD), jnp.float32),
    ],
    scratch_types=[
        pltpu.VMEM((CHUNK_SUBS, SUB), jnp.int32),
        pltpu.VMEM((SUB, DD), jnp.float32),
        pltpu.VMEM_SHARED((N_PAD, DD), jnp.float32),
        pltpu.SemaphoreType.DMA,
    ],
)(_deg_body)


def _inv_deg(c_ref):
    c = c_ref[0, :, 0:1] + c_ref[1, :, 0:1]
    return 1.0 / jnp.maximum(c, 1.0)


def _l0_paper_body(sw_ref, sc_ref, cw_ref, cc_ref, xp_ref,
                   wlw_ref, wlc_ref, wrw_ref, wrc_ref, b_ref, o_ref):
    aggw = (sw_ref[0] + sw_ref[1]) * _inv_deg(cw_ref)
    aggc = (sc_ref[0] + sc_ref[1]) * _inv_deg(cc_ref)
    h = jnp.dot(aggw, wlw_ref[...], preferred_element_type=jnp.float32)
    h = h + jnp.dot(aggc, wlc_ref[...], preferred_element_type=jnp.float32)
    h = h + jnp.dot(xp_ref[...], wrw_ref[...] + wrc_ref[...],
                    preferred_element_type=jnp.float32)
    o_ref[...] = jnp.maximum(h + b_ref[...], 0.0)


def _l0_author_body(sr_ref, cr_ref, xa_ref, wlr_ref, wrr_ref, b_ref, o_ref):
    aggr = (sr_ref[0] + sr_ref[1]) * _inv_deg(cr_ref)
    h = jnp.dot(aggr, wlr_ref[...], preferred_element_type=jnp.float32)
    h = h + jnp.dot(xa_ref[...], wrr_ref[...], preferred_element_type=jnp.float32)
    o_ref[...] = jnp.maximum(h + b_ref[...], 0.0)


def _mm_body(x_ref, w_ref, o_ref):
    o_ref[...] = jnp.dot(x_ref[...], w_ref[...], preferred_element_type=jnp.float32)


def _l1_paper_body(sw_ref, sc_ref, cw_ref, cc_ref, hp_ref,
                   wrw_ref, wrc_ref, b_ref, o_ref):
    aggw = (sw_ref[0] + sw_ref[1]) * _inv_deg(cw_ref)
    aggc = (sc_ref[0] + sc_ref[1]) * _inv_deg(cc_ref)
    h = aggw + aggc + jnp.dot(hp_ref[...], wrw_ref[...] + wrc_ref[...],
                              preferred_element_type=jnp.float32)
    o_ref[...] = h + b_ref[...]


def _part_spec(d):
    return pl.BlockSpec((NC, RT, d), lambda i: (0, i, 0))


def _row_spec(d):
    return pl.BlockSpec((RT, d), lambda i: (i, 0))


def _full_spec(r, d):
    return pl.BlockSpec((r, d), lambda i: (0, 0))


_GRID = (NN // RT,)

_l0_paper = pl.pallas_call(
    _l0_paper_body,
    grid=_GRID,
    in_specs=[_part_spec(DD), _part_spec(DD), _part_spec(DD), _part_spec(DD),
              _row_spec(DD), _full_spec(DD, 256), _full_spec(DD, 256),
              _full_spec(DD, 256), _full_spec(DD, 256), _full_spec(1, 256)],
    out_specs=_row_spec(256),
    out_shape=jax.ShapeDtypeStruct((NN, 256), jnp.float32),
)

_l0_author = pl.pallas_call(
    _l0_author_body,
    grid=_GRID,
    in_specs=[_part_spec(DD), _part_spec(DD), _row_spec(DD),
              _full_spec(DD, 256), _full_spec(DD, 256), _full_spec(1, 256)],
    out_specs=_row_spec(256),
    out_shape=jax.ShapeDtypeStruct((NN, 256), jnp.float32),
)

_mm_256_128 = pl.pallas_call(
    _mm_body,
    grid=_GRID,
    in_specs=[_row_spec(256), _full_spec(256, DD)],
    out_specs=_row_spec(DD),
    out_shape=jax.ShapeDtypeStruct((NN, DD), jnp.float32),
)

_l1_paper = pl.pallas_call(
    _l1_paper_body,
    grid=_GRID,
    in_specs=[_part_spec(DD), _part_spec(DD), _part_spec(DD), _part_spec(DD),
              _row_spec(256), _full_spec(256, DD), _full_spec(256, DD),
              _full_spec(1, DD)],
    out_specs=_row_spec(DD),
    out_shape=jax.ShapeDtypeStruct((NN, DD), jnp.float32),
)


def _prep_edges(ei):
    pad = E_PAD - EE
    src = jnp.concatenate([ei[0], jnp.zeros((pad,), jnp.int32)])
    dst = jnp.concatenate([ei[1], jnp.full((pad,), NN, jnp.int32)])
    return src.reshape(E_PAD // SUB, SUB), dst.reshape(E_PAD // SUB, SUB)


def kernel(x_paper, x_author, edge_index_writes, edge_index_cites,
           edge_index_rev_writes,
           Wl_writes_0, bl_writes_0, Wr_writes_0,
           Wl_cites_0, bl_cites_0, Wr_cites_0,
           Wl_rev_writes_0, bl_rev_writes_0, Wr_rev_writes_0,
           Wl_writes_1, bl_writes_1, Wr_writes_1,
           Wl_cites_1, bl_cites_1, Wr_cites_1,
           Wl_rev_writes_1, bl_rev_writes_1, Wr_rev_writes_1):
    zf = jnp.zeros((N_PAD, DD), jnp.float32)
    ones_n = jnp.ones((NN, DD), jnp.float32)

    src_w, dst_w = _prep_edges(edge_index_writes)
    src_c, dst_c = _prep_edges(edge_index_cites)
    src_r, dst_r = _prep_edges(edge_index_rev_writes)

    s_w0 = _seg_nocount(x_author, src_w, dst_w, zf)
    s_c0 = _seg_nocount(x_paper, src_c, dst_c, zf)
    s_r0 = _seg_nocount(x_paper, src_r, dst_r, zf)
    cnt_w, cnt_c, cnt_r = _deg_counts(ones_n, dst_w, dst_c, dst_r, zf)

    b_p0 = (bl_writes_0 + bl_cites_0).reshape(1, 256)
    b_a0 = bl_rev_writes_0.reshape(1, 256)
    h_p = _l0_paper(s_w0, s_c0, cnt_w, cnt_c, x_paper,
                    Wl_writes_0, Wl_cites_0, Wr_writes_0, Wr_cites_0, b_p0)
    h_a = _l0_author(s_r0, cnt_r, x_author, Wl_rev_writes_0, Wr_rev_writes_0,
                     b_a0)

    a1 = _mm_256_128(h_a, Wl_writes_1)
    pc = _mm_256_128(h_p, Wl_cites_1)
    s_w1 = _seg_nocount(a1, src_w, dst_w, zf)
    s_c1 = _seg_nocount(pc, src_c, dst_c, zf)

    b_p1 = (bl_writes_1 + bl_cites_1).reshape(1, DD)
    out = _l1_paper(s_w1, s_c1, cnt_w, cnt_c, h_p,
                    Wr_writes_1, Wr_cites_1, b_p1)
    return out


# speedup vs baseline: 3.6473x; 1.2645x over previous
import functools

import jax
import jax.numpy as jnp
from jax import lax
from jax.experimental import pallas as pl
from jax.experimental.pallas import tpu as pltpu
from jax.experimental.pallas import tpu_sc as plsc

NN = 10000
DD = 128
EE = 160000
NC = 2
NS = 16
SUB = 128
EW = 10240
E_PAD = EW * NS
CHUNK_SUBS = 8
N_CHUNK = EW // (SUB * CHUNK_SUBS)
N_PAD = 10112
ZROWS = N_PAD // NS
FROWS = 632
FLAST = NN - (NS - 1) * FROWS
RT = 1000


def _seg_mesh():
    return plsc.VectorSubcoreMesh(core_axis_name="c", subcore_axis_name="s")


def _hbm_to_spmem(hbm, spm, base, total, stage):
    off = 0
    while off < total:
        L = min(SUB, total - off)
        pltpu.sync_copy(hbm.at[pl.ds(base + off, L)], stage.at[pl.ds(0, L)])
        pltpu.sync_copy(stage.at[pl.ds(0, L)], spm.at[pl.ds(base + off, L)])
        off += L


def _spmem_to_hbm(spm, hbm, base, total, stage):
    off = 0
    while off < total:
        L = min(SUB, total - off)
        pltpu.sync_copy(spm.at[pl.ds(base + off, L)], stage.at[pl.ds(0, L)])
        pltpu.sync_copy(stage.at[pl.ds(0, L)], hbm.at[pl.ds(base + off, L)])
        off += L


def _zero_acc(s, zf_hbm, acc, stage):
    _hbm_to_spmem(zf_hbm, acc, s * ZROWS, ZROWS, stage)


def _flush_acc(s, acc, out_hbm, stage):
    f0 = s * FROWS

    @pl.when(s < NS - 1)
    def _flush_main():
        _spmem_to_hbm(acc, out_hbm, f0, FROWS, stage)

    @pl.when(s == NS - 1)
    def _flush_last():
        _spmem_to_hbm(acc, out_hbm, f0, FLAST, stage)


def _sum_job(s, x_hbm, src_hbm, dst_hbm, out_hbm, zf_hbm,
             idx_s, idx_d, rows, acc, sg, ss):
    _zero_acc(s, zf_hbm, acc, rows[1])
    plsc.subcore_barrier()
    row_base = s * (EW // SUB)

    def chunk(j, carry):
        r0 = row_base + j * CHUNK_SUBS
        pltpu.sync_copy(src_hbm.at[pl.ds(r0, CHUNK_SUBS)], idx_s)
        pltpu.sync_copy(dst_hbm.at[pl.ds(r0, CHUNK_SUBS)], idx_d)
        gd = [None, None]
        sd = [None, None]
        gd[0] = pltpu.async_copy(x_hbm.at[idx_s.at[0]], rows[0], sg[0])
        for i in range(CHUNK_SUBS):
            cb = i % 2
            gd[cb].wait()
            if i < CHUNK_SUBS - 1:
                if i >= 1:
                    sd[1 - cb].wait()
                gd[1 - cb] = pltpu.async_copy(
                    x_hbm.at[idx_s.at[i + 1]], rows[1 - cb], sg[1 - cb])
            sd[cb] = pltpu.async_copy(
                rows[cb], acc.at[idx_d.at[i]], ss[cb], add=True)
        sd[0].wait()
        sd[1].wait()
        return carry

    lax.fori_loop(0, N_CHUNK, chunk, 0)
    plsc.subcore_barrier()
    _flush_acc(s, acc, out_hbm, rows[1])
    plsc.subcore_barrier()


def _count_job(s, dst_hbm, out_hbm, zf_hbm, idx_d, rows, acc, ss):
    _zero_acc(s, zf_hbm, acc, rows[1])
    plsc.subcore_barrier()
    row_base = s * (EW // SUB)

    def chunk(j, carry):
        r0 = row_base + j * CHUNK_SUBS
        pltpu.sync_copy(dst_hbm.at[pl.ds(r0, CHUNK_SUBS)], idx_d)
        ds = [pltpu.async_copy(rows[0], acc.at[idx_d.at[i]], ss[i % 2],
                               add=True)
              for i in range(CHUNK_SUBS)]
        for d in ds:
            d.wait()
        return carry

    lax.fori_loop(0, N_CHUNK, chunk, 0)
    plsc.subcore_barrier()
    _flush_acc(s, acc, out_hbm, rows[1])
    plsc.subcore_barrier()


def _l0_sc_body(xp_hbm, xa_hbm, srcw, dstw, srcc, dstc, srcr, dstr,
                zf_hbm, ones_hbm,
                sw_out, sc_out, sr_out, cw_out, cc_out, cr_out,
                idx_s, idx_d, rows0, rows1, acc, sg0, sg1, ss0, ss1):
    c = lax.axis_index("c")
    s = lax.axis_index("s")
    rows = (rows0, rows1)
    sg = (sg0, sg1)
    ss = (ss0, ss1)

    pltpu.sync_copy(ones_hbm, rows0)

    @pl.when(c == 0)
    def _core0_counts():
        _count_job(s, dstr, cr_out, zf_hbm, idx_d, rows, acc, ss)

    @pl.when(c == 1)
    def _core1_counts():
        _count_job(s, dstw, cw_out, zf_hbm, idx_d, rows, acc, ss)
        _count_job(s, dstc, cc_out, zf_hbm, idx_d, rows, acc, ss)

    @pl.when(c == 0)
    def _core0_sums():
        _sum_job(s, xa_hbm, srcw, dstw, sw_out, zf_hbm,
                 idx_s, idx_d, rows, acc, sg, ss)
        _sum_job(s, xp_hbm, srcc, dstc, sc_out, zf_hbm,
                 idx_s, idx_d, rows, acc, sg, ss)

    @pl.when(c == 1)
    def _core1_sums():
        _sum_job(s, xp_hbm, srcr, dstr, sr_out, zf_hbm,
                 idx_s, idx_d, rows, acc, sg, ss)


def _l1_sc_body(a1_hbm, pc_hbm, srcw, dstw, srcc, dstc, zf_hbm,
                sw_out, sc_out,
                idx_s, idx_d, rows0, rows1, acc, sg0, sg1, ss0, ss1):
    c = lax.axis_index("c")
    s = lax.axis_index("s")
    rows = (rows0, rows1)
    sg = (sg0, sg1)
    ss = (ss0, ss1)

    @pl.when(c == 0)
    def _core0():
        _sum_job(s, a1_hbm, srcw, dstw, sw_out, zf_hbm,
                 idx_s, idx_d, rows, acc, sg, ss)

    @pl.when(c == 1)
    def _core1():
        _sum_job(s, pc_hbm, srcc, dstc, sc_out, zf_hbm,
                 idx_s, idx_d, rows, acc, sg, ss)


_SC_SCRATCH = [
    pltpu.VMEM((CHUNK_SUBS, SUB), jnp.int32),
    pltpu.VMEM((CHUNK_SUBS, SUB), jnp.int32),
    pltpu.VMEM((SUB, DD), jnp.float32),
    pltpu.VMEM((SUB, DD), jnp.float32),
    pltpu.VMEM_SHARED((N_PAD, DD), jnp.float32),
    pltpu.SemaphoreType.DMA,
    pltpu.SemaphoreType.DMA,
    pltpu.SemaphoreType.DMA,
    pltpu.SemaphoreType.DMA,
]

_l0_sc = functools.partial(
    pl.kernel,
    mesh=_seg_mesh(),
    out_type=[jax.ShapeDtypeStruct((NN, DD), jnp.float32)] * 6,
    scratch_types=_SC_SCRATCH,
)(_l0_sc_body)

_l1_sc = functools.partial(
    pl.kernel,
    mesh=_seg_mesh(),
    out_type=[jax.ShapeDtypeStruct((NN, DD), jnp.float32)] * 2,
    scratch_types=_SC_SCRATCH,
)(_l1_sc_body)


def _inv_deg(c_ref):
    return 1.0 / jnp.maximum(c_ref[:, 0:1], 1.0)


def _l0_tc_body(sw_ref, sc_ref, sr_ref, cw_ref, cc_ref, cr_ref,
                xp_ref, xa_ref,
                wlw_ref, wlc_ref, wlr_ref, wrw_ref, wrc_ref, wrr_ref,
                bp_ref, ba_ref, wlw1_ref, wlc1_ref,
                hp_ref, ha_ref, a1_ref, pc_ref):
    aggw = sw_ref[...] * _inv_deg(cw_ref)
    aggc = sc_ref[...] * _inv_deg(cc_ref)
    aggr = sr_ref[...] * _inv_deg(cr_ref)
    hp = jnp.dot(aggw, wlw_ref[...], preferred_element_type=jnp.float32)
    hp = hp + jnp.dot(aggc, wlc_ref[...], preferred_element_type=jnp.float32)
    hp = hp + jnp.dot(xp_ref[...], wrw_ref[...] + wrc_ref[...],
                      preferred_element_type=jnp.float32)
    hp = jnp.maximum(hp + bp_ref[...], 0.0)
    ha = jnp.dot(aggr, wlr_ref[...], preferred_element_type=jnp.float32)
    ha = ha + jnp.dot(xa_ref[...], wrr_ref[...],
                      preferred_element_type=jnp.float32)
    ha = jnp.maximum(ha + ba_ref[...], 0.0)
    hp_ref[...] = hp
    ha_ref[...] = ha
    a1_ref[...] = jnp.dot(ha, wlw1_ref[...], preferred_element_type=jnp.float32)
    pc_ref[...] = jnp.dot(hp, wlc1_ref[...], preferred_element_type=jnp.float32)


def _l1_tc_body(sw_ref, sc_ref, cw_ref, cc_ref, hp_ref,
                wrw_ref, wrc_ref, b_ref, o_ref):
    aggw = sw_ref[...] * _inv_deg(cw_ref)
    aggc = sc_ref[...] * _inv_deg(cc_ref)
    h = aggw + aggc + jnp.dot(hp_ref[...], wrw_ref[...] + wrc_ref[...],
                              preferred_element_type=jnp.float32)
    o_ref[...] = h + b_ref[...]


def _row_spec(d):
    return pl.BlockSpec((RT, d), lambda i: (i, 0))


def _full_spec(r, d):
    return pl.BlockSpec((r, d), lambda i: (0, 0))


_GRID = (NN // RT,)

_l0_tc = pl.pallas_call(
    _l0_tc_body,
    grid=_GRID,
    in_specs=[_row_spec(DD)] * 6 + [_row_spec(DD)] * 2
    + [_full_spec(DD, 256)] * 6 + [_full_spec(1, 256)] * 2
    + [_full_spec(256, DD)] * 2,
    out_specs=[_row_spec(256), _row_spec(256), _row_spec(DD), _row_spec(DD)],
    out_shape=[
        jax.ShapeDtypeStruct((NN, 256), jnp.float32),
        jax.ShapeDtypeStruct((NN, 256), jnp.float32),
        jax.ShapeDtypeStruct((NN, DD), jnp.float32),
        jax.ShapeDtypeStruct((NN, DD), jnp.float32),
    ],
)

_l1_tc = pl.pallas_call(
    _l1_tc_body,
    grid=_GRID,
    in_specs=[_row_spec(DD), _row_spec(DD), _row_spec(DD), _row_spec(DD),
              _row_spec(256), _full_spec(256, DD), _full_spec(256, DD),
              _full_spec(1, DD)],
    out_specs=_row_spec(DD),
    out_shape=jax.ShapeDtypeStruct((NN, DD), jnp.float32),
)


def _prep_edges(ei):
    pad = E_PAD - EE
    src = jnp.concatenate([ei[0], jnp.zeros((pad,), jnp.int32)])
    dst = jnp.concatenate([ei[1], jnp.full((pad,), NN, jnp.int32)])
    return src.reshape(E_PAD // SUB, SUB), dst.reshape(E_PAD // SUB, SUB)


def kernel(x_paper, x_author, edge_index_writes, edge_index_cites,
           edge_index_rev_writes,
           Wl_writes_0, bl_writes_0, Wr_writes_0,
           Wl_cites_0, bl_cites_0, Wr_cites_0,
           Wl_rev_writes_0, bl_rev_writes_0, Wr_rev_writes_0,
           Wl_writes_1, bl_writes_1, Wr_writes_1,
           Wl_cites_1, bl_cites_1, Wr_cites_1,
           Wl_rev_writes_1, bl_rev_writes_1, Wr_rev_writes_1):
    zf = jnp.zeros((N_PAD, DD), jnp.float32)
    ones = jnp.ones((SUB, DD), jnp.float32)

    src_w, dst_w = _prep_edges(edge_index_writes)
    src_c, dst_c = _prep_edges(edge_index_cites)
    src_r, dst_r = _prep_edges(edge_index_rev_writes)

    s_w0, s_c0, s_r0, cnt_w, cnt_c, cnt_r = _l0_sc(
        x_paper, x_author, src_w, dst_w, src_c, dst_c, src_r, dst_r,
        zf, ones)

    b_p0 = (bl_writes_0 + bl_cites_0).reshape(1, 256)
    b_a0 = bl_rev_writes_0.reshape(1, 256)
    h_p, h_a, a1, pc = _l0_tc(
        s_w0, s_c0, s_r0, cnt_w, cnt_c, cnt_r, x_paper, x_author,
        Wl_writes_0, Wl_cites_0, Wl_rev_writes_0,
        Wr_writes_0, Wr_cites_0, Wr_rev_writes_0,
        b_p0, b_a0, Wl_writes_1, Wl_cites_1)

    s_w1, s_c1 = _l1_sc(a1, pc, src_w, dst_w, src_c, dst_c, zf)

    b_p1 = (bl_writes_1 + bl_cites_1).reshape(1, DD)
    out = _l1_tc(s_w1, s_c1, cnt_w, cnt_c, h_p,
                 Wr_writes_1, Wr_cites_1, b_p1)
    return out


# speedup vs baseline: 3.7829x; 1.0372x over previous
import functools

import jax
import jax.numpy as jnp
from jax import lax
from jax.experimental import pallas as pl
from jax.experimental.pallas import tpu as pltpu
from jax.experimental.pallas import tpu_sc as plsc

NN = 10000
DD = 128
EE = 160000
NC = 2
NS = 16
SUB = 128
EW = 10240
E_PAD = EW * NS
CHUNK_SUBS = 8
N_CHUNK = EW // (SUB * CHUNK_SUBS)
NSUBS = EW // SUB
RB = 64
NBUF = 5
NBLK = CHUNK_SUBS * SUB // RB
N_PAD = 10112
ZROWS = N_PAD // NS
FROWS = 632
FLAST = NN - (NS - 1) * FROWS
RT = 1000


def _seg_mesh():
    return plsc.VectorSubcoreMesh(core_axis_name="c", subcore_axis_name="s")


def _hbm_to_spmem(hbm, spm, base, total, stage):
    off = 0
    while off < total:
        L = min(RB, total - off)
        pltpu.sync_copy(hbm.at[pl.ds(base + off, L)], stage.at[pl.ds(0, L)])
        pltpu.sync_copy(stage.at[pl.ds(0, L)], spm.at[pl.ds(base + off, L)])
        off += L


def _spmem_to_hbm(spm, hbm, base, total, stage):
    off = 0
    while off < total:
        L = min(RB, total - off)
        pltpu.sync_copy(spm.at[pl.ds(base + off, L)], stage.at[pl.ds(0, L)])
        pltpu.sync_copy(stage.at[pl.ds(0, L)], hbm.at[pl.ds(base + off, L)])
        off += L


def _zero_acc(s, zf_hbm, acc, stage):
    _hbm_to_spmem(zf_hbm, acc, s * ZROWS, ZROWS, stage)


def _flush_acc(s, acc, out_hbm, stage):
    f0 = s * FROWS

    @pl.when(s < NS - 1)
    def _flush_main():
        _spmem_to_hbm(acc, out_hbm, f0, FROWS, stage)

    @pl.when(s == NS - 1)
    def _flush_last():
        _spmem_to_hbm(acc, out_hbm, f0, FLAST, stage)


def _blk(idx, k):
    return idx.at[k // 2, pl.ds((k % 2) * RB, RB)]


def _sum_job(s, x_hbm, src_hbm, dst_hbm, out_hbm, zf_hbm,
             idx_s, idx_d, rows, acc, sg, ss):
    _zero_acc(s, zf_hbm, acc, rows[1])
    row_base = s * NSUBS
    plsc.subcore_barrier()

    def chunk(j, carry):
        r0 = row_base + j * CHUNK_SUBS
        pltpu.sync_copy(src_hbm.at[pl.ds(r0, CHUNK_SUBS)], idx_s)
        pltpu.sync_copy(dst_hbm.at[pl.ds(r0, CHUNK_SUBS)], idx_d)
        gd = [None] * NBUF
        sd = [None] * NBUF
        for k in range(NBUF):
            gd[k] = pltpu.async_copy(x_hbm.at[_blk(idx_s, k)],
                                     rows[k], sg[k])
        for k in range(NBLK):
            b = k % NBUF
            gd[b].wait()
            sd[b] = pltpu.async_copy(
                rows[b], acc.at[_blk(idx_d, k)], ss[b], add=True)
            if k + NBUF < NBLK:
                sd[b].wait()
                gd[b] = pltpu.async_copy(
                    x_hbm.at[_blk(idx_s, k + NBUF)], rows[b], sg[b])
        for k in range(NBLK - NBUF, NBLK):
            sd[k % NBUF].wait()
        return carry

    lax.fori_loop(0, N_CHUNK, chunk, 0)
    plsc.subcore_barrier()
    _flush_acc(s, acc, out_hbm, rows[1])
    plsc.subcore_barrier()


def _count_job(s, dst_hbm, out_hbm, zf_hbm, idx_d, rows, acc, ss):
    _zero_acc(s, zf_hbm, acc, rows[1])
    row_base = s * NSUBS
    plsc.subcore_barrier()

    def chunk(j, carry):
        r0 = row_base + j * CHUNK_SUBS
        pltpu.sync_copy(dst_hbm.at[pl.ds(r0, CHUNK_SUBS)], idx_d)
        sd = [None] * NBUF
        for k in range(NBLK):
            b = k % NBUF
            if k >= NBUF:
                sd[b].wait()
            sd[b] = pltpu.async_copy(rows[0], acc.at[_blk(idx_d, k)],
                                     ss[b], add=True)
        for b in range(NBUF):
            sd[b].wait()
        return carry

    lax.fori_loop(0, N_CHUNK, chunk, 0)
    plsc.subcore_barrier()
    _flush_acc(s, acc, out_hbm, rows[1])
    plsc.subcore_barrier()


def _l0_sc_body(xp_hbm, xa_hbm, srcw, dstw, srcc, dstc, srcr, dstr,
                zf_hbm, ones_hbm,
                sw_out, sc_out, sr_out, cw_out, cc_out, cr_out,
                idx_s, idx_d, rows0, rows1, rows2, rows3, rows4, acc,
                sg0, sg1, sg2, sg3, sg4, ss0, ss1, ss2, ss3, ss4):
    c = lax.axis_index("c")
    s = lax.axis_index("s")
    rows = (rows0, rows1, rows2, rows3, rows4)
    sg = (sg0, sg1, sg2, sg3, sg4)
    ss = (ss0, ss1, ss2, ss3, ss4)

    pltpu.sync_copy(ones_hbm, rows0)

    @pl.when(c == 0)
    def _core0_counts():
        _count_job(s, dstr, cr_out, zf_hbm, idx_d, rows, acc, ss)

    @pl.when(c == 1)
    def _core1_counts():
        _count_job(s, dstw, cw_out, zf_hbm, idx_d, rows, acc, ss)
        _count_job(s, dstc, cc_out, zf_hbm, idx_d, rows, acc, ss)

    @pl.when(c == 0)
    def _core0_sums():
        _sum_job(s, xa_hbm, srcw, dstw, sw_out, zf_hbm,
                 idx_s, idx_d, rows, acc, sg, ss)
        _sum_job(s, xp_hbm, srcc, dstc, sc_out, zf_hbm,
                 idx_s, idx_d, rows, acc, sg, ss)

    @pl.when(c == 1)
    def _core1_sums():
        _sum_job(s, xp_hbm, srcr, dstr, sr_out, zf_hbm,
                 idx_s, idx_d, rows, acc, sg, ss)


def _l1_sc_body(a1_hbm, pc_hbm, srcw, dstw, srcc, dstc, zf_hbm,
                sw_out, sc_out,
                idx_s, idx_d, rows0, rows1, rows2, rows3, rows4, acc,
                sg0, sg1, sg2, sg3, sg4, ss0, ss1, ss2, ss3, ss4):
    c = lax.axis_index("c")
    s = lax.axis_index("s")
    rows = (rows0, rows1, rows2, rows3, rows4)
    sg = (sg0, sg1, sg2, sg3, sg4)
    ss = (ss0, ss1, ss2, ss3, ss4)

    @pl.when(c == 0)
    def _core0():
        _sum_job(s, a1_hbm, srcw, dstw, sw_out, zf_hbm,
                 idx_s, idx_d, rows, acc, sg, ss)

    @pl.when(c == 1)
    def _core1():
        _sum_job(s, pc_hbm, srcc, dstc, sc_out, zf_hbm,
                 idx_s, idx_d, rows, acc, sg, ss)


_SC_SCRATCH = [
    pltpu.VMEM((CHUNK_SUBS, SUB), jnp.int32),
    pltpu.VMEM((CHUNK_SUBS, SUB), jnp.int32),
] + [pltpu.VMEM((RB, DD), jnp.float32)] * NBUF + [
    pltpu.VMEM_SHARED((N_PAD, DD), jnp.float32),
] + [pltpu.SemaphoreType.DMA] * (2 * NBUF)

_l0_sc = functools.partial(
    pl.kernel,
    mesh=_seg_mesh(),
    out_type=[jax.ShapeDtypeStruct((NN, DD), jnp.float32)] * 6,
    scratch_types=_SC_SCRATCH,
)(_l0_sc_body)

_l1_sc = functools.partial(
    pl.kernel,
    mesh=_seg_mesh(),
    out_type=[jax.ShapeDtypeStruct((NN, DD), jnp.float32)] * 2,
    scratch_types=_SC_SCRATCH,
)(_l1_sc_body)


def _inv_deg(c_ref):
    return 1.0 / jnp.maximum(c_ref[:, 0:1], 1.0)


def _l0_tc_body(sw_ref, sc_ref, sr_ref, cw_ref, cc_ref, cr_ref,
                xp_ref, xa_ref,
                wlw_ref, wlc_ref, wlr_ref, wrw_ref, wrc_ref, wrr_ref,
                bp_ref, ba_ref, wlw1_ref, wlc1_ref,
                hp_ref, ha_ref, a1_ref, pc_ref):
    aggw = sw_ref[...] * _inv_deg(cw_ref)
    aggc = sc_ref[...] * _inv_deg(cc_ref)
    aggr = sr_ref[...] * _inv_deg(cr_ref)
    hp = jnp.dot(aggw, wlw_ref[...], preferred_element_type=jnp.float32)
    hp = hp + jnp.dot(aggc, wlc_ref[...], preferred_element_type=jnp.float32)
    hp = hp + jnp.dot(xp_ref[...], wrw_ref[...] + wrc_ref[...],
                      preferred_element_type=jnp.float32)
    hp = jnp.maximum(hp + bp_ref[...], 0.0)
    ha = jnp.dot(aggr, wlr_ref[...], preferred_element_type=jnp.float32)
    ha = ha + jnp.dot(xa_ref[...], wrr_ref[...],
                      preferred_element_type=jnp.float32)
    ha = jnp.maximum(ha + ba_ref[...], 0.0)
    hp_ref[...] = hp
    ha_ref[...] = ha
    a1_ref[...] = jnp.dot(ha, wlw1_ref[...], preferred_element_type=jnp.float32)
    pc_ref[...] = jnp.dot(hp, wlc1_ref[...], preferred_element_type=jnp.float32)


def _l1_tc_body(sw_ref, sc_ref, cw_ref, cc_ref, hp_ref,
                wrw_ref, wrc_ref, b_ref, o_ref):
    aggw = sw_ref[...] * _inv_deg(cw_ref)
    aggc = sc_ref[...] * _inv_deg(cc_ref)
    h = aggw + aggc + jnp.dot(hp_ref[...], wrw_ref[...] + wrc_ref[...],
                              preferred_element_type=jnp.float32)
    o_ref[...] = h + b_ref[...]


def _row_spec(d):
    return pl.BlockSpec((RT, d), lambda i: (i, 0))


def _full_spec(r, d):
    return pl.BlockSpec((r, d), lambda i: (0, 0))


_GRID = (NN // RT,)

_l0_tc = pl.pallas_call(
    _l0_tc_body,
    grid=_GRID,
    in_specs=[_row_spec(DD)] * 6 + [_row_spec(DD)] * 2
    + [_full_spec(DD, 256)] * 6 + [_full_spec(1, 256)] * 2
    + [_full_spec(256, DD)] * 2,
    out_specs=[_row_spec(256), _row_spec(256), _row_spec(DD), _row_spec(DD)],
    out_shape=[
        jax.ShapeDtypeStruct((NN, 256), jnp.float32),
        jax.ShapeDtypeStruct((NN, 256), jnp.float32),
        jax.ShapeDtypeStruct((NN, DD), jnp.float32),
        jax.ShapeDtypeStruct((NN, DD), jnp.float32),
    ],
)

_l1_tc = pl.pallas_call(
    _l1_tc_body,
    grid=_GRID,
    in_specs=[_row_spec(DD), _row_spec(DD), _row_spec(DD), _row_spec(DD),
              _row_spec(256), _full_spec(256, DD), _full_spec(256, DD),
              _full_spec(1, DD)],
    out_specs=_row_spec(DD),
    out_shape=jax.ShapeDtypeStruct((NN, DD), jnp.float32),
)


def _prep_edges(ei):
    pad = E_PAD - EE
    src = jnp.concatenate([ei[0], jnp.zeros((pad,), jnp.int32)])
    dst = jnp.concatenate([ei[1], jnp.full((pad,), NN, jnp.int32)])
    return src.reshape(E_PAD // SUB, SUB), dst.reshape(E_PAD // SUB, SUB)


def kernel(x_paper, x_author, edge_index_writes, edge_index_cites,
           edge_index_rev_writes,
           Wl_writes_0, bl_writes_0, Wr_writes_0,
           Wl_cites_0, bl_cites_0, Wr_cites_0,
           Wl_rev_writes_0, bl_rev_writes_0, Wr_rev_writes_0,
           Wl_writes_1, bl_writes_1, Wr_writes_1,
           Wl_cites_1, bl_cites_1, Wr_cites_1,
           Wl_rev_writes_1, bl_rev_writes_1, Wr_rev_writes_1):
    zf = jnp.zeros((N_PAD, DD), jnp.float32)
    ones = jnp.ones((RB, DD), jnp.float32)

    src_w, dst_w = _prep_edges(edge_index_writes)
    src_c, dst_c = _prep_edges(edge_index_cites)
    src_r, dst_r = _prep_edges(edge_index_rev_writes)

    s_w0, s_c0, s_r0, cnt_w, cnt_c, cnt_r = _l0_sc(
        x_paper, x_author, src_w, dst_w, src_c, dst_c, src_r, dst_r,
        zf, ones)

    b_p0 = (bl_writes_0 + bl_cites_0).reshape(1, 256)
    b_a0 = bl_rev_writes_0.reshape(1, 256)
    h_p, h_a, a1, pc = _l0_tc(
        s_w0, s_c0, s_r0, cnt_w, cnt_c, cnt_r, x_paper, x_author,
        Wl_writes_0, Wl_cites_0, Wl_rev_writes_0,
        Wr_writes_0, Wr_cites_0, Wr_rev_writes_0,
        b_p0, b_a0, Wl_writes_1, Wl_cites_1)

    s_w1, s_c1 = _l1_sc(a1, pc, src_w, dst_w, src_c, dst_c, zf)

    b_p1 = (bl_writes_1 + bl_cites_1).reshape(1, DD)
    out = _l1_tc(s_w1, s_c1, cnt_w, cnt_c, h_p,
                 Wr_writes_1, Wr_cites_1, b_p1)
    return out


# speedup vs baseline: 3.9063x; 1.0326x over previous
import functools

import jax
import jax.numpy as jnp
from jax import lax
from jax.experimental import pallas as pl
from jax.experimental.pallas import tpu as pltpu
from jax.experimental.pallas import tpu_sc as plsc

NN = 10000
DD = 128
EE = 160000
NC = 2
NS = 16
SUB = 128
EW = 10240
E_PAD = EW * NS
CHUNK_SUBS = 8
N_CHUNK = EW // (SUB * CHUNK_SUBS)
NSUBS = EW // SUB
RB = 64
NBUF = 5
NBLK = CHUNK_SUBS * SUB // RB
N_PAD = 10112
ZROWS = N_PAD // NS
FROWS = 632
FLAST = NN - (NS - 1) * FROWS
RT = 1000


def _seg_mesh():
    return plsc.VectorSubcoreMesh(core_axis_name="c", subcore_axis_name="s")


def _hbm_to_spmem(hbm, spm, base, total, stage):
    off = 0
    while off < total:
        L = min(RB, total - off)
        pltpu.sync_copy(hbm.at[pl.ds(base + off, L)], stage.at[pl.ds(0, L)])
        pltpu.sync_copy(stage.at[pl.ds(0, L)], spm.at[pl.ds(base + off, L)])
        off += L


def _spmem_to_hbm(spm, hbm, base, total, stage):
    off = 0
    while off < total:
        L = min(RB, total - off)
        pltpu.sync_copy(spm.at[pl.ds(base + off, L)], stage.at[pl.ds(0, L)])
        pltpu.sync_copy(stage.at[pl.ds(0, L)], hbm.at[pl.ds(base + off, L)])
        off += L


def _zero_acc(s, zf_hbm, acc, stage):
    _hbm_to_spmem(zf_hbm, acc, s * ZROWS, ZROWS, stage)


def _flush_acc(s, acc, out_hbm, stage):
    f0 = s * FROWS

    @pl.when(s < NS - 1)
    def _flush_main():
        _spmem_to_hbm(acc, out_hbm, f0, FROWS, stage)

    @pl.when(s == NS - 1)
    def _flush_last():
        _spmem_to_hbm(acc, out_hbm, f0, FLAST, stage)


def _blk(idx, k):
    return idx.at[k // 2, pl.ds((k % 2) * RB, RB)]


def _sum_job(s, x_hbm, src_hbm, dst_hbm, out_hbm, zf_hbm,
             idx_s, idx_d, rows, acc, sg, ss,
             row_base=None, n_chunk=N_CHUNK):
    _zero_acc(s, zf_hbm, acc, rows[1])
    if row_base is None:
        row_base = s * NSUBS
    plsc.subcore_barrier()

    def chunk(j, carry):
        r0 = row_base + j * CHUNK_SUBS
        pltpu.sync_copy(src_hbm.at[pl.ds(r0, CHUNK_SUBS)], idx_s)
        pltpu.sync_copy(dst_hbm.at[pl.ds(r0, CHUNK_SUBS)], idx_d)
        gd = [None] * NBUF
        sd = [None] * NBUF
        for k in range(NBUF):
            gd[k] = pltpu.async_copy(x_hbm.at[_blk(idx_s, k)],
                                     rows[k], sg[k])
        for k in range(NBLK):
            b = k % NBUF
            gd[b].wait()
            sd[b] = pltpu.async_copy(
                rows[b], acc.at[_blk(idx_d, k)], ss[b], add=True)
            if k + NBUF < NBLK:
                sd[b].wait()
                gd[b] = pltpu.async_copy(
                    x_hbm.at[_blk(idx_s, k + NBUF)], rows[b], sg[b])
        for k in range(NBLK - NBUF, NBLK):
            sd[k % NBUF].wait()
        return carry

    lax.fori_loop(0, n_chunk, chunk, 0)
    plsc.subcore_barrier()
    _flush_acc(s, acc, out_hbm, rows[1])
    plsc.subcore_barrier()


def _count_job(s, dst_hbm, out_hbm, zf_hbm, idx_d, rows, acc, ss,
               row_base=None, n_chunk=N_CHUNK):
    _zero_acc(s, zf_hbm, acc, rows[1])
    if row_base is None:
        row_base = s * NSUBS
    plsc.subcore_barrier()

    def chunk(j, carry):
        r0 = row_base + j * CHUNK_SUBS
        pltpu.sync_copy(dst_hbm.at[pl.ds(r0, CHUNK_SUBS)], idx_d)
        sd = [None] * NBUF
        for k in range(NBLK):
            b = k % NBUF
            if k >= NBUF:
                sd[b].wait()
            sd[b] = pltpu.async_copy(rows[0], acc.at[_blk(idx_d, k)],
                                     ss[b], add=True)
        for b in range(NBUF):
            sd[b].wait()
        return carry

    lax.fori_loop(0, n_chunk, chunk, 0)
    plsc.subcore_barrier()
    _flush_acc(s, acc, out_hbm, rows[1])
    plsc.subcore_barrier()


HALF_ROWS = E_PAD // SUB // 2
NSUBS_H = HALF_ROWS // NS
N_CHUNK_H = NSUBS_H // CHUNK_SUBS


def _l0_sc_body(xp_hbm, xa_hbm, srcw, dstw, srcc, dstc, srcr, dstr,
                zf_hbm, ones_hbm,
                sw_out, sc0_out, sc1_out, sr_out,
                cw_out, cc0_out, cc1_out, cr_out,
                idx_s, idx_d, rows0, rows1, rows2, rows3, rows4, acc,
                sg0, sg1, sg2, sg3, sg4, ss0, ss1, ss2, ss3, ss4):
    c = lax.axis_index("c")
    s = lax.axis_index("s")
    rows = (rows0, rows1, rows2, rows3, rows4)
    sg = (sg0, sg1, sg2, sg3, sg4)
    ss = (ss0, ss1, ss2, ss3, ss4)
    half_base = c * HALF_ROWS + s * NSUBS_H

    pltpu.sync_copy(ones_hbm, rows0)

    @pl.when(c == 0)
    def _core0():
        _count_job(s, dstw, cw_out, zf_hbm, idx_d, rows, acc, ss)
        _count_job(s, dstc, cc0_out, zf_hbm, idx_d, rows, acc, ss,
                   row_base=half_base, n_chunk=N_CHUNK_H)
        _sum_job(s, xa_hbm, srcw, dstw, sw_out, zf_hbm,
                 idx_s, idx_d, rows, acc, sg, ss)
        _sum_job(s, xp_hbm, srcc, dstc, sc0_out, zf_hbm,
                 idx_s, idx_d, rows, acc, sg, ss,
                 row_base=half_base, n_chunk=N_CHUNK_H)

    @pl.when(c == 1)
    def _core1():
        _count_job(s, dstr, cr_out, zf_hbm, idx_d, rows, acc, ss)
        _count_job(s, dstc, cc1_out, zf_hbm, idx_d, rows, acc, ss,
                   row_base=half_base, n_chunk=N_CHUNK_H)
        _sum_job(s, xp_hbm, srcr, dstr, sr_out, zf_hbm,
                 idx_s, idx_d, rows, acc, sg, ss)
        _sum_job(s, xp_hbm, srcc, dstc, sc1_out, zf_hbm,
                 idx_s, idx_d, rows, acc, sg, ss,
                 row_base=half_base, n_chunk=N_CHUNK_H)


def _l1_sc_body(a1_hbm, pc_hbm, srcw, dstw, srcc, dstc, zf_hbm,
                sw_out, sc_out,
                idx_s, idx_d, rows0, rows1, rows2, rows3, rows4, acc,
                sg0, sg1, sg2, sg3, sg4, ss0, ss1, ss2, ss3, ss4):
    c = lax.axis_index("c")
    s = lax.axis_index("s")
    rows = (rows0, rows1, rows2, rows3, rows4)
    sg = (sg0, sg1, sg2, sg3, sg4)
    ss = (ss0, ss1, ss2, ss3, ss4)

    @pl.when(c == 0)
    def _core0():
        _sum_job(s, a1_hbm, srcw, dstw, sw_out, zf_hbm,
                 idx_s, idx_d, rows, acc, sg, ss)

    @pl.when(c == 1)
    def _core1():
        _sum_job(s, pc_hbm, srcc, dstc, sc_out, zf_hbm,
                 idx_s, idx_d, rows, acc, sg, ss)


_SC_SCRATCH = [
    pltpu.VMEM((CHUNK_SUBS, SUB), jnp.int32),
    pltpu.VMEM((CHUNK_SUBS, SUB), jnp.int32),
] + [pltpu.VMEM((RB, DD), jnp.float32)] * NBUF + [
    pltpu.VMEM_SHARED((N_PAD, DD), jnp.float32),
] + [pltpu.SemaphoreType.DMA] * (2 * NBUF)

_l0_sc = functools.partial(
    pl.kernel,
    mesh=_seg_mesh(),
    out_type=[jax.ShapeDtypeStruct((NN, DD), jnp.float32)] * 8,
    scratch_types=_SC_SCRATCH,
)(_l0_sc_body)

_l1_sc = functools.partial(
    pl.kernel,
    mesh=_seg_mesh(),
    out_type=[jax.ShapeDtypeStruct((NN, DD), jnp.float32)] * 2,
    scratch_types=_SC_SCRATCH,
)(_l1_sc_body)


def _inv_deg(c_ref):
    return 1.0 / jnp.maximum(c_ref[:, 0:1], 1.0)


def _l0_tc_body(sw_ref, sc0_ref, sc1_ref, sr_ref,
                cw_ref, cc0_ref, cc1_ref, cr_ref,
                xp_ref, xa_ref,
                wlw_ref, wlc_ref, wlr_ref, wrw_ref, wrc_ref, wrr_ref,
                bp_ref, ba_ref, wlw1_ref, wlc1_ref,
                hp_ref, ha_ref, a1_ref, pc_ref):
    aggw = sw_ref[...] * _inv_deg(cw_ref)
    cc = jnp.maximum(cc0_ref[:, 0:1] + cc1_ref[:, 0:1], 1.0)
    aggc = (sc0_ref[...] + sc1_ref[...]) * (1.0 / cc)
    aggr = sr_ref[...] * _inv_deg(cr_ref)
    hp = jnp.dot(aggw, wlw_ref[...], preferred_element_type=jnp.float32)
    hp = hp + jnp.dot(aggc, wlc_ref[...], preferred_element_type=jnp.float32)
    hp = hp + jnp.dot(xp_ref[...], wrw_ref[...] + wrc_ref[...],
                      preferred_element_type=jnp.float32)
    hp = jnp.maximum(hp + bp_ref[...], 0.0)
    ha = jnp.dot(aggr, wlr_ref[...], preferred_element_type=jnp.float32)
    ha = ha + jnp.dot(xa_ref[...], wrr_ref[...],
                      preferred_element_type=jnp.float32)
    ha = jnp.maximum(ha + ba_ref[...], 0.0)
    hp_ref[...] = hp
    ha_ref[...] = ha
    a1_ref[...] = jnp.dot(ha, wlw1_ref[...], preferred_element_type=jnp.float32)
    pc_ref[...] = jnp.dot(hp, wlc1_ref[...], preferred_element_type=jnp.float32)


def _l1_tc_body(sw_ref, sc_ref, cw_ref, cc0_ref, cc1_ref, hp_ref,
                wrw_ref, wrc_ref, b_ref, o_ref):
    aggw = sw_ref[...] * _inv_deg(cw_ref)
    cc = jnp.maximum(cc0_ref[:, 0:1] + cc1_ref[:, 0:1], 1.0)
    aggc = sc_ref[...] * (1.0 / cc)
    h = aggw + aggc + jnp.dot(hp_ref[...], wrw_ref[...] + wrc_ref[...],
                              preferred_element_type=jnp.float32)
    o_ref[...] = h + b_ref[...]


def _row_spec(d):
    return pl.BlockSpec((RT, d), lambda i: (i, 0))


def _full_spec(r, d):
    return pl.BlockSpec((r, d), lambda i: (0, 0))


_GRID = (NN // RT,)

_l0_tc = pl.pallas_call(
    _l0_tc_body,
    grid=_GRID,
    in_specs=[_row_spec(DD)] * 8 + [_row_spec(DD)] * 2
    + [_full_spec(DD, 256)] * 6 + [_full_spec(1, 256)] * 2
    + [_full_spec(256, DD)] * 2,
    out_specs=[_row_spec(256), _row_spec(256), _row_spec(DD), _row_spec(DD)],
    out_shape=[
        jax.ShapeDtypeStruct((NN, 256), jnp.float32),
        jax.ShapeDtypeStruct((NN, 256), jnp.float32),
        jax.ShapeDtypeStruct((NN, DD), jnp.float32),
        jax.ShapeDtypeStruct((NN, DD), jnp.float32),
    ],
)

_l1_tc = pl.pallas_call(
    _l1_tc_body,
    grid=_GRID,
    in_specs=[_row_spec(DD), _row_spec(DD), _row_spec(DD), _row_spec(DD),
              _row_spec(DD), _row_spec(256), _full_spec(256, DD),
              _full_spec(256, DD), _full_spec(1, DD)],
    out_specs=_row_spec(DD),
    out_shape=jax.ShapeDtypeStruct((NN, DD), jnp.float32),
)


def _prep_edges(ei):
    pad = E_PAD - EE
    src = jnp.concatenate([ei[0], jnp.zeros((pad,), jnp.int32)])
    dst = jnp.concatenate([ei[1], jnp.full((pad,), NN, jnp.int32)])
    return src.reshape(E_PAD // SUB, SUB), dst.reshape(E_PAD // SUB, SUB)


def kernel(x_paper, x_author, edge_index_writes, edge_index_cites,
           edge_index_rev_writes,
           Wl_writes_0, bl_writes_0, Wr_writes_0,
           Wl_cites_0, bl_cites_0, Wr_cites_0,
           Wl_rev_writes_0, bl_rev_writes_0, Wr_rev_writes_0,
           Wl_writes_1, bl_writes_1, Wr_writes_1,
           Wl_cites_1, bl_cites_1, Wr_cites_1,
           Wl_rev_writes_1, bl_rev_writes_1, Wr_rev_writes_1):
    zf = jnp.zeros((N_PAD, DD), jnp.float32)
    ones = jnp.ones((RB, DD), jnp.float32)

    src_w, dst_w = _prep_edges(edge_index_writes)
    src_c, dst_c = _prep_edges(edge_index_cites)
    src_r, dst_r = _prep_edges(edge_index_rev_writes)

    s_w0, s_ca, s_cb, s_r0, cnt_w, cnt_ca, cnt_cb, cnt_r = _l0_sc(
        x_paper, x_author, src_w, dst_w, src_c, dst_c, src_r, dst_r,
        zf, ones)

    b_p0 = (bl_writes_0 + bl_cites_0).reshape(1, 256)
    b_a0 = bl_rev_writes_0.reshape(1, 256)
    h_p, h_a, a1, pc = _l0_tc(
        s_w0, s_ca, s_cb, s_r0, cnt_w, cnt_ca, cnt_cb, cnt_r,
        x_paper, x_author,
        Wl_writes_0, Wl_cites_0, Wl_rev_writes_0,
        Wr_writes_0, Wr_cites_0, Wr_rev_writes_0,
        b_p0, b_a0, Wl_writes_1, Wl_cites_1)

    s_w1, s_c1 = _l1_sc(a1, pc, src_w, dst_w, src_c, dst_c, zf)

    b_p1 = (bl_writes_1 + bl_cites_1).reshape(1, DD)
    out = _l1_tc(s_w1, s_c1, cnt_w, cnt_ca, cnt_cb, h_p,
                 Wr_writes_1, Wr_cites_1, b_p1)
    return out
